# baseline (speedup 1.0000x reference)
import jax
import jax.numpy as jnp
from jax.experimental import pallas as pl
from jax.experimental.pallas import tpu as pltpu

_B, _T, _F, _H, _S = 32, 2048, 64, 512, 64
_TC = 64
_NC = _T // _TC
_R = _TC * _B
_SUB = 256
_NSUB = _R // _SUB
_SPT = _SUB // _B
_EPS = 1e-5


class _LayerCtx:
    def __init__(self, wrefs, gbufs, read_rows, write_rows, st0):
        (self.lng, self.lnb, self.xpW, self.xpb, self.dpW, self.dpb,
         self.At, self.Ct, self.gW, self.gb, self.oW, self.ob) = wrefs
        self.gbufs = gbufs
        self.read_rows = read_rows
        self.write_rows = write_rows
        self.st = st0
        self.xn_v = [None] * _NSUB
        self.xp_v = [None] * _NSUB
        self.dl_v = [None] * _NSUB
        self.sts_v = [None] * _NSUB
        self.st_sl = [None] * _TC

    def ln_op(self, s):
        rows = self.read_rows(s * _SUB, _SUB)
        mu = jnp.mean(rows, axis=-1, keepdims=True)
        ctr = rows - mu
        var = jnp.mean(ctr * ctr, axis=-1, keepdims=True)
        xn = (ctr * jax.lax.rsqrt(var + _EPS) * self.lng[...]
              + self.lnb[...])
        self.xn_v[s] = xn.astype(jnp.bfloat16)

    def xp_op(self, s):
        self.xp_v[s] = (jnp.dot(self.xn_v[s], self.xpW[...],
                                preferred_element_type=jnp.float32)
                        + self.xpb[...])

    def dl_op(self, s):
        self.dl_v[s] = jax.nn.sigmoid(
            jnp.dot(self.xn_v[s], self.dpW[...],
                    preferred_element_type=jnp.float32) + self.dpb[...])

    def gate_op(self, s):
        g = jax.nn.sigmoid(
            jnp.dot(self.xn_v[s], self.gW[...],
                    preferred_element_type=jnp.float32) + self.gb[...])
        self.gbufs[s % 2][...] = g.astype(jnp.bfloat16)

    def post_op(self, s):
        sC = jnp.dot(self.sts_v[s], self.Ct[...],
                     preferred_element_type=jnp.float32)
        prod = self.gbufs[s % 2][...] * sC.astype(jnp.bfloat16)
        out = jnp.dot(prod, self.oW[...], preferred_element_type=jnp.float32)
        res = self.read_rows(s * _SUB, _SUB)
        self.write_rows(s * _SUB, out + self.ob[...] + res)

    def step(self, t):
        s, k = divmod(t, _SPT)
        if k == 0 and s >= 1:
            self.post_op(s - 1)
        if s + 1 < _NSUB:
            if k == 1:
                self.ln_op(s + 1)
            elif k == 2:
                self.xp_op(s + 1)
            elif k == 3:
                self.dl_op(s + 1)
            elif k == 4:
                self.gate_op(s + 1)
        xt = self.xp_v[s][k * _B:(k + 1) * _B, :]
        dt = self.dl_v[s][k * _B:(k + 1) * _B, :]
        sA = jnp.dot(self.st, self.At[...], preferred_element_type=jnp.float32)
        self.st = (1.0 - dt) * self.st + dt * (sA + xt)
        self.st_sl[t] = self.st
        if k == _SPT - 1:
            self.sts_v[s] = jnp.concatenate(
                self.st_sl[s * _SPT:(s + 1) * _SPT],
                axis=0).astype(jnp.bfloat16)


_NL = 2


def _quad_body(h_ref, *rest):
    ws = [rest[12 * i:12 * (i + 1)] for i in range(_NL)]
    tail = rest[12 * _NL:]
    o_ref = tail[0]
    rings = tail[1:_NL]
    states = tail[_NL:2 * _NL]
    gbufs = tail[2 * _NL:]
    j = pl.program_id(0)

    def reader(i):
        if i == 0:
            return lambda r0, n: h_ref[r0:r0 + n, :]
        p = jax.lax.rem(j + i, 2)
        return lambda r0, n: rings[i - 1][p, r0:r0 + n, :]

    def writer(i):
        if i == _NL - 1:
            return lambda r0, v: o_ref.__setitem__(
                (pl.ds(r0, v.shape[0]), slice(None)), v)
        p = jax.lax.rem(j + i, 2)
        return lambda r0, v: rings[i].__setitem__(
            (p, pl.ds(r0, v.shape[0]), slice(None)), v)

    ctxs = [None] * _NL
    for i in range(_NL - 1, -1, -1):
        ctxs[i] = _LayerCtx(
            ws[i], (gbufs[2 * i], gbufs[2 * i + 1]),
            reader(i), writer(i),
            jnp.where(j > i, states[i][...], 0.0))
    order = list(range(_NL - 1, -1, -1))
    for i in order:
        ctx = ctxs[i]
        ctx.ln_op(0)
        ctx.xp_op(0)
        ctx.dl_op(0)
        ctx.gate_op(0)
    for t in range(_TC):
        for i in order:
            ctxs[i].step(t)
    for i in order:
        ctxs[i].post_op(_NSUB - 1)
    for i in range(_NL):
        states[i][...] = ctxs[i].st


def _quad(h2, wlists):
    full = lambda s: pl.BlockSpec(s, lambda j: (0,) * len(s))
    wspecs = [
        full((1, _H)), full((1, _H)),
        full((_H, _S)), full((1, _S)),
        full((_H, _S)), full((1, _S)),
        full((_S, _S)), full((_S, _H)),
        full((_H, _H)), full((1, _H)),
        full((_H, _H)), full((1, _H)),
    ]
    flat_w = [w for wl in wlists for w in wl]
    return pl.pallas_call(
        _quad_body,
        grid=(_NC + _NL - 1,),
        in_specs=[pl.BlockSpec((_R, _H),
                               lambda j: (jnp.minimum(j, _NC - 1), 0))]
                 + wspecs * _NL,
        out_specs=pl.BlockSpec(
            (_R, _H), lambda j: (jnp.maximum(j - (_NL - 1), 0), 0)),
        out_shape=jax.ShapeDtypeStruct((_T * _B, _H), jnp.float32),
        scratch_shapes=(
            [pltpu.VMEM((2, _R, _H), jnp.float32)] * (_NL - 1)
            + [pltpu.VMEM((_B, _S), jnp.float32)] * _NL
            + [pltpu.VMEM((_SUB, _H), jnp.bfloat16)] * (2 * _NL)
        ),
        compiler_params=pltpu.CompilerParams(
            dimension_semantics=("arbitrary",),
            vmem_limit_bytes=56 * 1024 * 1024,
        ),
        name="mamba_quad",
    )(h2, *flat_w)


def _inproj_body(x_ref, w_ref, b_ref, o_ref):
    o_ref[...] = (jnp.dot(x_ref[...], w_ref[...],
                          preferred_element_type=jnp.float32) + b_ref[...])


def _inproj(xt, inW, inb):
    rows = 4096
    return pl.pallas_call(
        _inproj_body,
        grid=(_T * _B // rows,),
        in_specs=[
            pl.BlockSpec((rows, _F), lambda j: (j, 0)),
            pl.BlockSpec((_F, _H), lambda j: (0, 0)),
            pl.BlockSpec((1, _H), lambda j: (0, 0)),
        ],
        out_specs=pl.BlockSpec((rows, _H), lambda j: (j, 0)),
        out_shape=jax.ShapeDtypeStruct((_T * _B, _H), jnp.float32),
        compiler_params=pltpu.CompilerParams(
            dimension_semantics=("parallel",),
            vmem_limit_bytes=56 * 1024 * 1024,
        ),
        name="mamba_inproj",
    )(xt, inW, inb)


def _erf(z):
    s = jnp.where(z < 0, -1.0, 1.0)
    a = jnp.abs(z)
    t = 1.0 / (1.0 + 0.3275911 * a)
    p = t * (0.254829592 + t * (-0.284496736 + t * (1.421413741
        + t * (-1.453152027 + t * 1.061405429))))
    return s * (1.0 - p * jnp.exp(-a * a))


def _head_body(h_ref, g_ref, b_ref, w1_ref, b1_ref, w2_ref, b2_ref, o_ref):
    rows = h_ref[...]
    mu = jnp.mean(rows, axis=-1, keepdims=True)
    ctr = rows - mu
    var = jnp.mean(ctr * ctr, axis=-1, keepdims=True)
    y = ctr * jax.lax.rsqrt(var + _EPS) * g_ref[...] + b_ref[...]
    y = jnp.dot(y, w1_ref[...], preferred_element_type=jnp.float32) + b1_ref[...]
    y = y * 0.5 * (1.0 + _erf(y * 0.7071067811865476))
    o_ref[...] = (jnp.dot(y, w2_ref[...], preferred_element_type=jnp.float32)
                  + b2_ref[...])


def _head(last, hln_g, hln_b, h1W, h1b, h2W, h2b):
    return pl.pallas_call(
        _head_body,
        out_shape=jax.ShapeDtypeStruct((_B, 1), jnp.float32),
        name="mamba_head",
    )(last, hln_g, hln_b, h1W, h1b, h2W, h2b)


def kernel(x, inW, inb, ln_g, ln_b, xpW, xpb, dpW, dpb, A, C, gW, gb,
           oW, ob, hln_g, hln_b, h1W, h1b, h2W, h2b):
    L = ln_g.shape[0]
    bf = jnp.bfloat16

    def wlist(i):
        return [ln_g[i].reshape(1, _H), ln_b[i].reshape(1, _H),
                xpW[i].astype(bf), xpb[i].reshape(1, _S),
                dpW[i].astype(bf), dpb[i].reshape(1, _S),
                A[i].T, C[i].T.astype(bf),
                gW[i].astype(bf), gb[i].reshape(1, _H),
                oW[i].astype(bf), ob[i].reshape(1, _H)]

    xt = jnp.transpose(x, (1, 0, 2)).reshape(_T * _B, _F)
    h = _inproj(xt, inW, inb.reshape(1, _H))
    for i in range(0, L, _NL):
        h = _quad(h, [wlist(i + d) for d in range(_NL)])
    last = h[-_B:, :]
    return _head(last, hln_g.reshape(1, _H), hln_b.reshape(1, _H),
                 h1W, h1b.reshape(1, _H // 2), h2W, h2b.reshape(1, 1))
